# baseline (speedup 1.0000x reference)
import functools

import jax
import jax.numpy as jnp
from jax import lax
from jax.experimental import pallas as pl
from jax.experimental.pallas import tpu as pltpu
from jax.experimental.pallas import tpu_sc as plsc

ROWS = 1035
WIN = 456
WOUT = 272
NBLK = WOUT // 16

NC = 2
NS = 16
NW = NC * NS
RPW = 33
MAXBASE = ROWS - RPW

SCALE = 1.6764705882352942
INV_SCALE = 0.5964912280701754

_MESH = plsc.VectorSubcoreMesh(
    core_axis_name="c", subcore_axis_name="s", num_cores=NC, num_subcores=NS
)


def _resize_body(in_hbm, out_hbm, in_v, out_v, idx_v, w_v):
    wid = lax.axis_index("s") * NC + lax.axis_index("c")
    base = jnp.minimum(wid * RPW, MAXBASE)

    pltpu.sync_copy(in_hbm.at[pl.ds(base * WIN, RPW * WIN)], in_v)

    for b in range(NBLK):
        i = lax.iota(jnp.int32, 16) + (b * 16)
        center = (i.astype(jnp.float32) + 0.5) * SCALE
        xmin = jnp.maximum((center - SCALE + 0.5).astype(jnp.int32), 0)
        xmax = jnp.minimum((center + SCALE + 0.5).astype(jnp.int32), WIN)
        ksize = jnp.minimum(xmax - xmin, 5)
        xmin_f = xmin.astype(jnp.float32)
        ws = []
        for j in range(5):
            dist = (xmin_f + float(j) - center + 0.5) * INV_SCALE
            wj = 1.0 - jnp.minimum(jnp.abs(dist), 1.0)
            ws.append(jnp.where(ksize > j, wj, 0.0))
        total = ws[0] + ws[1] + ws[2] + ws[3] + ws[4]
        for j in range(4):
            w_v[pl.ds(j * WOUT + b * 16, 16)] = ws[j] / total
            idx_v[pl.ds(j * WOUT + b * 16, 16)] = jnp.minimum(xmin + j, WIN - 1)

    for b in range(NBLK):
        idxs = [idx_v[pl.ds(j * WOUT + b * 16, 16)] for j in range(4)]
        wgts = [w_v[pl.ds(j * WOUT + b * 16, 16)] for j in range(4)]

        def row_body(r, carry, idxs=idxs, wgts=wgts, b=b):
            roff = r * WIN
            acc = wgts[0] * plsc.load_gather(in_v, [idxs[0] + roff])
            for j in range(1, 4):
                acc += wgts[j] * plsc.load_gather(in_v, [idxs[j] + roff])
            out_v[pl.ds(r * WOUT + b * 16, 16)] = acc
            return carry

        lax.fori_loop(0, RPW, row_body, 0)

    pltpu.sync_copy(out_v, out_hbm.at[pl.ds(base * WOUT, RPW * WOUT)])


_resize = pl.kernel(
    _resize_body,
    out_type=jax.ShapeDtypeStruct((ROWS * WOUT,), jnp.float32),
    mesh=_MESH,
    compiler_params=pltpu.CompilerParams(needs_layout_passes=False),
    scratch_types=[
        pltpu.VMEM((RPW * WIN,), jnp.float32),
        pltpu.VMEM((RPW * WOUT,), jnp.float32),
        pltpu.VMEM((4 * WOUT,), jnp.int32),
        pltpu.VMEM((4 * WOUT,), jnp.float32),
    ],
)


@jax.jit
def kernel(arg0_1):
    flat = arg0_1.reshape(ROWS * WIN)
    out = _resize(flat)
    return (out.reshape(1, 3, 345, WOUT),)
